# baseline (speedup 1.0000x reference)
import functools

import jax
import jax.numpy as jnp
from jax.experimental import pallas as pl
from jax.experimental.pallas import tpu as pltpu

_TVS = 512


def _topk_kern(x_ref, wt_ref, b_ref, out_ref, best, vidx, *, v_total, tvs):
    j = pl.program_id(0)
    nj = pl.num_programs(0)

    rs = jnp.sum(wt_ref[...] * x_ref[...], axis=1, keepdims=True)
    score = jnp.transpose(rs, (1, 0)) + b_ref[...]
    lane = jax.lax.broadcasted_iota(jnp.int32, score.shape, 1)
    gv = j * tvs + lane
    score = jnp.where(gv < v_total, score, -jnp.inf)

    @pl.when(j == 0)
    def _init():
        best[...] = score
        vidx[...] = gv

    @pl.when(j > 0)
    def _update():
        upd = score > best[...]
        best[...] = jnp.where(upd, score, best[...])
        vidx[...] = jnp.where(upd, gv, vidx[...])

    @pl.when(j == nj - 1)
    def _emit():
        b_all = best[...]
        m = jnp.max(b_all)
        out_ref[0] = jnp.min(
            jnp.where(b_all == m, vidx[...], jnp.iinfo(jnp.int32).max)
        )


def kernel(x, W, b):
    d, v = W.shape
    tvs = min(_TVS, v)
    nj = pl.cdiv(v, tvs)
    wt = W.T
    x0 = x[0:1]
    b2 = b.reshape(1, v)
    out = pl.pallas_call(
        functools.partial(_topk_kern, v_total=v, tvs=tvs),
        grid=(nj,),
        in_specs=[
            pl.BlockSpec((1, d), lambda j: (0, 0)),
            pl.BlockSpec((tvs, d), lambda j: (j, 0)),
            pl.BlockSpec((1, tvs), lambda j: (0, j)),
        ],
        out_specs=pl.BlockSpec(memory_space=pltpu.SMEM),
        out_shape=jax.ShapeDtypeStruct((1,), jnp.int32),
        scratch_shapes=[
            pltpu.VMEM((1, tvs), jnp.float32),
            pltpu.VMEM((1, tvs), jnp.int32),
        ],
        compiler_params=pltpu.CompilerParams(
            dimension_semantics=("arbitrary",),
        ),
    )(x0, wt, b2)
    return out


# speedup vs baseline: 1.1366x; 1.1366x over previous
import functools

import jax
import jax.numpy as jnp
from jax.experimental import pallas as pl
from jax.experimental.pallas import tpu as pltpu

_TVS = 1024


def _topk_kern(x_ref, wt_ref, b_ref, out_ref, best_val, best_idx, *, v_total, tvs):
    j = pl.program_id(0)
    nj = pl.num_programs(0)

    @pl.when(j == 0)
    def _init():
        best_val[0] = -jnp.inf
        best_idx[0] = 0

    rs = jnp.sum(wt_ref[...] * x_ref[...], axis=1, keepdims=True)
    score = jnp.transpose(rs, (1, 0)) + b_ref[...]
    lane = jax.lax.broadcasted_iota(jnp.int32, score.shape, 1)
    gv = j * tvs + lane
    score = jnp.where(gv < v_total, score, -jnp.inf)
    m = jnp.max(score)

    @pl.when(m > best_val[0])
    def _update():
        best_val[0] = m
        best_idx[0] = jnp.min(
            jnp.where(score == m, gv, jnp.iinfo(jnp.int32).max)
        )

    @pl.when(j == nj - 1)
    def _emit():
        out_ref[0] = best_idx[0]


def kernel(x, W, b):
    d, v = W.shape
    tvs = min(_TVS, v)
    nj = pl.cdiv(v, tvs)
    wt = W.T
    x0 = x[0:1]
    b2 = b.reshape(1, v)
    out = pl.pallas_call(
        functools.partial(_topk_kern, v_total=v, tvs=tvs),
        grid=(nj,),
        in_specs=[
            pl.BlockSpec((1, d), lambda j: (0, 0)),
            pl.BlockSpec((tvs, d), lambda j: (j, 0)),
            pl.BlockSpec((1, tvs), lambda j: (0, j)),
        ],
        out_specs=pl.BlockSpec(memory_space=pltpu.SMEM),
        out_shape=jax.ShapeDtypeStruct((1,), jnp.int32),
        scratch_shapes=[
            pltpu.SMEM((1,), jnp.float32),
            pltpu.SMEM((1,), jnp.int32),
        ],
        compiler_params=pltpu.CompilerParams(
            dimension_semantics=("arbitrary",),
        ),
    )(x0, wt, b2)
    return out
